# Initial kernel scaffold; baseline (speedup 1.0000x reference)
import jax
import jax.numpy as jnp
from jax.experimental import pallas as pl


def kernel(idx, pe):
    raise NotImplementedError("write your pallas kernel here")


# speedup vs baseline: 3.9364x; 3.9364x over previous
import functools

import jax
import jax.numpy as jnp
from jax import lax
from jax.experimental import pallas as pl
from jax.experimental.pallas import tpu as pltpu
from jax.experimental.pallas import tpu_sc as plsc

_K = 128
_G = 4
_CHUNK = _K * _G


@functools.cache
def _build(B, V, D):
    info = plsc.get_sparse_core_info()
    NC, NS = info.num_cores, info.num_subcores
    NW = NC * NS
    assert B % (NW * _CHUNK) == 0
    b_per_w = B // NW
    steps = b_per_w // _CHUNK
    k_rows_per_w = b_per_w // _K

    mesh = plsc.VectorSubcoreMesh(core_axis_name="c", subcore_axis_name="s")

    @functools.partial(
        pl.kernel,
        out_type=jax.ShapeDtypeStruct((B, D), jnp.float32),
        mesh=mesh,
        scratch_types=[
            pltpu.VMEM((k_rows_per_w, _K), jnp.int32),
            pltpu.VMEM((_CHUNK, D), jnp.float32),
            pltpu.SemaphoreType.DMA,
        ],
        compiler_params=pltpu.CompilerParams(use_tc_tiling_on_sc=False),
    )
    def gather_kernel(idx_hbm, table_hbm, out_hbm, idx_v, rows_v, sem):
        wid = lax.axis_index("s") * NC + lax.axis_index("c")
        base = wid * b_per_w
        pltpu.sync_copy(idx_hbm.at[pl.ds(wid * k_rows_per_w, k_rows_per_w)],
                        idx_v)

        @pl.loop(0, steps)
        def _(g):
            descs = []
            for j in range(_G):
                descs.append(pltpu.async_copy(
                    table_hbm.at[idx_v.at[g * _G + j]],
                    rows_v.at[pl.ds(j * _K, _K)],
                    sem))
            for d in descs:
                d.wait()
            pltpu.sync_copy(rows_v,
                            out_hbm.at[pl.ds(base + g * _CHUNK, _CHUNK)])

    return gather_kernel


def kernel(idx, pe):
    B = idx.size
    V, D = pe.shape
    idx_flat = idx.reshape(B // _K, _K).astype(jnp.int32)
    out = _build(B, V, D)(idx_flat, pe)
    return out.reshape(idx.shape + (D,))


# speedup vs baseline: 4.1009x; 1.0418x over previous
import functools

import jax
import jax.numpy as jnp
from jax import lax
from jax.experimental import pallas as pl
from jax.experimental.pallas import tpu as pltpu
from jax.experimental.pallas import tpu_sc as plsc

_K = 128
_G = 4
_CHUNK = _K * _G


@functools.cache
def _build(B, V, D):
    info = plsc.get_sparse_core_info()
    NC, NS = info.num_cores, info.num_subcores
    NW = NC * NS
    assert B % (NW * _CHUNK) == 0
    b_per_w = B // NW
    steps = b_per_w // _CHUNK
    k_rows_per_w = b_per_w // _K

    mesh = plsc.VectorSubcoreMesh(core_axis_name="c", subcore_axis_name="s")

    @functools.partial(
        pl.kernel,
        out_type=jax.ShapeDtypeStruct((B, D), jnp.float32),
        mesh=mesh,
        scratch_types=[
            pltpu.VMEM((k_rows_per_w, _K), jnp.int32),
            pltpu.VMEM((_CHUNK, D), jnp.float32),
            pltpu.VMEM((_CHUNK, D), jnp.float32),
            pltpu.SemaphoreType.DMA,
            pltpu.SemaphoreType.DMA,
        ],
        compiler_params=pltpu.CompilerParams(use_tc_tiling_on_sc=False),
    )
    def gather_kernel(idx_hbm, table_hbm, out_hbm, idx_v, b0, b1, s0, s1):
        wid = lax.axis_index("s") * NC + lax.axis_index("c")
        base = wid * b_per_w
        pltpu.sync_copy(idx_hbm.at[pl.ds(wid * k_rows_per_w, k_rows_per_w)],
                        idx_v)

        def fire(chunk, buf, sem):
            return [pltpu.async_copy(
                table_hbm.at[idx_v.at[chunk * _G + j]],
                buf.at[pl.ds(j * _K, _K)], sem) for j in range(_G)]

        def drain(buf, sem):
            pltpu.make_async_copy(out_hbm.at[pl.ds(0, _CHUNK)], buf,
                                  sem).wait()

        fire(0, b0, s0)

        @pl.loop(0, steps, step=2)
        def _(g):
            fire(g + 1, b1, s1)
            drain(b0, s0)
            pltpu.sync_copy(b0, out_hbm.at[pl.ds(base + g * _CHUNK, _CHUNK)])

            @pl.when(g + 2 < steps)
            def _():
                fire(g + 2, b0, s0)

            drain(b1, s1)
            pltpu.sync_copy(
                b1, out_hbm.at[pl.ds(base + (g + 1) * _CHUNK, _CHUNK)])

    return gather_kernel


def kernel(idx, pe):
    B = idx.size
    V, D = pe.shape
    idx_flat = idx.reshape(B // _K, _K).astype(jnp.int32)
    out = _build(B, V, D)(idx_flat, pe)
    return out.reshape(idx.shape + (D,))
